# baseline (device time: 49750 ns/iter reference)
import jax
import jax.numpy as jnp
from jax import lax
from jax.experimental import pallas as pl
from jax.experimental.pallas import tpu as pltpu

N_DEV = 32
S = 4
G = 16
HALF = 512
CH = 32


def kernel(A, B):
    m, k_loc = A.shape
    _, n = B.shape
    w = n // S

    def body(a_ref, b_ref, out_ref, partial_ref, xrecv_ref, hbf_ref,
             yzrecv_ref, red32_ref, redbf_ref, halfbuf_ref,
             xrs_s, xrs_r, yz1_s, yz1_r, yz2_s, yz2_r, xag_s, xag_r):
        me = lax.axis_index("i")
        p = me // 8
        q = me % 8
        y = q // 2
        r4 = q % 4
        x = ((r4 % 2) + (r4 // 2)) % 2
        g = p * 4 + y
        my_off = x * HALF
        other_off = (1 - x) * HALF
        xp_id = p * 8 + q + 1 - 2 * (q % 2)

        def yz_peer(go):
            pp = go // 4
            yy = go % 4
            par = yy % 2
            qq = 2 * yy + x + par * (1 - 2 * x)
            return pp * 8 + qq

        barrier = pltpu.get_barrier_semaphore()
        peer_ids = [xp_id] + [yz_peer((g + o) % G) for o in range(1, G)]
        for pid in peer_ids:
            pl.semaphore_signal(
                barrier, inc=1,
                device_id=(pid,), device_id_type=pl.DeviceIdType.MESH,
            )
        bv = b_ref[:, :].astype(jnp.bfloat16)
        av_other = a_ref[pl.ds(other_off, HALF), :].astype(jnp.bfloat16)
        part_other = jnp.dot(av_other, bv, preferred_element_type=jnp.float32)
        partial_ref[pl.ds(other_off, HALF), :] = part_other.astype(jnp.bfloat16)

        pl.semaphore_wait(barrier, len(peer_ids))

        sends = []

        xrs = []
        for s in range(S):
            cols = pl.ds(s * w, w)
            rdma = pltpu.make_async_remote_copy(
                src_ref=partial_ref.at[pl.ds(other_off, HALF), cols],
                dst_ref=xrecv_ref.at[:, cols],
                send_sem=xrs_s.at[s],
                recv_sem=xrs_r.at[s],
                device_id=(xp_id,),
                device_id_type=pl.DeviceIdType.MESH,
            )
            rdma.start()
            xrs.append(rdma)
            sends.append(rdma)

        av_mine = a_ref[pl.ds(my_off, HALF), :].astype(jnp.bfloat16)
        part_mine = jnp.dot(av_mine, bv, preferred_element_type=jnp.float32)
        partial_ref[pl.ds(my_off, HALF), :] = part_mine.astype(jnp.bfloat16)

        def stage1_strip(s):
            cols = pl.ds(s * w, w)
            xrs[s].wait_recv()
            hbf_ref[:, cols] = (
                partial_ref[pl.ds(my_off, HALF), cols].astype(jnp.float32)
                + xrecv_ref[:, cols].astype(jnp.float32)
            ).astype(jnp.bfloat16)
            for o in range(1, G):
                tg = (g + o) % G
                rdma = pltpu.make_async_remote_copy(
                    src_ref=hbf_ref.at[pl.ds(tg * CH, CH), cols],
                    dst_ref=yzrecv_ref.at[pl.ds(g * CH, CH), cols],
                    send_sem=yz1_s.at[o - 1, s],
                    recv_sem=yz1_r.at[o - 1, s],
                    device_id=(yz_peer(tg),),
                    device_id_type=pl.DeviceIdType.MESH,
                )
                rdma.start()
                sends.append(rdma)

        def stage2_strip(s):
            cols = pl.ds(s * w, w)
            red32_ref[:, cols] = hbf_ref[pl.ds(g * CH, CH), cols].astype(
                jnp.float32
            )
            for o in range(1, G):
                sg = (g + o) % G
                wr = pltpu.make_async_remote_copy(
                    src_ref=hbf_ref.at[pl.ds(0, CH), cols],
                    dst_ref=yzrecv_ref.at[pl.ds(sg * CH, CH), cols],
                    send_sem=yz1_s.at[o - 1, s],
                    recv_sem=yz1_r.at[(G - o) - 1, s],
                    device_id=(yz_peer(sg),),
                    device_id_type=pl.DeviceIdType.MESH,
                )
                wr.wait_recv()
                red32_ref[:, cols] = red32_ref[:, cols] + yzrecv_ref[
                    pl.ds(sg * CH, CH), cols
                ].astype(jnp.float32)
            redbf_ref[:, cols] = red32_ref[:, cols].astype(jnp.bfloat16)
            for o in range(1, G):
                tg = (g + o) % G
                rdma = pltpu.make_async_remote_copy(
                    src_ref=redbf_ref.at[:, cols],
                    dst_ref=halfbuf_ref.at[pl.ds(g * CH, CH), cols],
                    send_sem=yz2_s.at[o - 1, s],
                    recv_sem=yz2_r.at[o - 1, s],
                    device_id=(yz_peer(tg),),
                    device_id_type=pl.DeviceIdType.MESH,
                )
                rdma.start()
                sends.append(rdma)
            halfbuf_ref[pl.ds(g * CH, CH), cols] = redbf_ref[:, cols]

        def stage3_strip(s):
            cols = pl.ds(s * w, w)
            for o in range(1, G):
                sg = (g + o) % G
                wr = pltpu.make_async_remote_copy(
                    src_ref=redbf_ref.at[:, cols],
                    dst_ref=halfbuf_ref.at[pl.ds(sg * CH, CH), cols],
                    send_sem=yz2_s.at[o - 1, s],
                    recv_sem=yz2_r.at[(G - o) - 1, s],
                    device_id=(yz_peer(sg),),
                    device_id_type=pl.DeviceIdType.MESH,
                )
                wr.wait_recv()
            rdma = pltpu.make_async_remote_copy(
                src_ref=halfbuf_ref.at[:, cols],
                dst_ref=out_ref.at[pl.ds(my_off, HALF), cols],
                send_sem=xag_s.at[s],
                recv_sem=xag_r.at[s],
                device_id=(xp_id,),
                device_id_type=pl.DeviceIdType.MESH,
            )
            rdma.start()
            sends.append(rdma)
            out_ref[pl.ds(my_off, HALF), cols] = halfbuf_ref[:, cols]

        for s in range(S):
            stage1_strip(s)
        for s in range(S):
            stage2_strip(s)
            if s >= 1:
                stage3_strip(s - 1)
        stage3_strip(S - 1)

        for s in range(S):
            cols = pl.ds(s * w, w)
            wr = pltpu.make_async_remote_copy(
                src_ref=halfbuf_ref.at[:, cols],
                dst_ref=out_ref.at[pl.ds(other_off, HALF), cols],
                send_sem=xag_s.at[s],
                recv_sem=xag_r.at[s],
                device_id=(xp_id,),
                device_id_type=pl.DeviceIdType.MESH,
            )
            wr.wait_recv()

        for rdma in sends:
            rdma.wait_send()

    return pl.pallas_call(
        body,
        out_shape=jax.ShapeDtypeStruct((m, n), jnp.bfloat16),
        in_specs=[
            pl.BlockSpec(memory_space=pltpu.VMEM),
            pl.BlockSpec(memory_space=pltpu.VMEM),
        ],
        out_specs=pl.BlockSpec(memory_space=pltpu.VMEM),
        scratch_shapes=[
            pltpu.VMEM((m, n), jnp.bfloat16),
            pltpu.VMEM((HALF, n), jnp.bfloat16),
            pltpu.VMEM((HALF, n), jnp.bfloat16),
            pltpu.VMEM((G * CH, n), jnp.bfloat16),
            pltpu.VMEM((CH, n), jnp.float32),
            pltpu.VMEM((CH, n), jnp.bfloat16),
            pltpu.VMEM((HALF, n), jnp.bfloat16),
            pltpu.SemaphoreType.DMA((S,)),
            pltpu.SemaphoreType.DMA((S,)),
            pltpu.SemaphoreType.DMA((G - 1, S)),
            pltpu.SemaphoreType.DMA((G - 1, S)),
            pltpu.SemaphoreType.DMA((G - 1, S)),
            pltpu.SemaphoreType.DMA((G - 1, S)),
            pltpu.SemaphoreType.DMA((S,)),
            pltpu.SemaphoreType.DMA((S,)),
        ],
        compiler_params=pltpu.CompilerParams(collective_id=0),
    )(A, B)
